# baseline (speedup 1.0000x reference)
import functools

import jax
import jax.numpy as jnp
from jax import lax
from jax.experimental import pallas as pl
from jax.experimental.pallas import tpu as pltpu
from jax.experimental.pallas import tpu_sc as plsc

N_NODES = 10000
D_FEAT = 128
N_PAIRS = 320000

NC = 2
NS = 16
NW = NC * NS
P_W = N_PAIRS // NW
CHUNK = 80
NCHUNK = P_W // CHUNK
BLKS = CHUNK // 16


def _make_sc_kernel():
    mesh = plsc.VectorSubcoreMesh(core_axis_name="c", subcore_axis_name="s")

    @functools.partial(
        pl.kernel,
        mesh=mesh,
        compiler_params=pltpu.CompilerParams(needs_layout_passes=False),
        out_type=jax.ShapeDtypeStruct((NW, 16), jnp.float32),
        scratch_types=[
            pltpu.VMEM((P_W,), jnp.int32),
            pltpu.VMEM((P_W,), jnp.int32),
            pltpu.VMEM((P_W,), jnp.int32),
            pltpu.VMEM((CHUNK, D_FEAT), jnp.float32),
            pltpu.VMEM((CHUNK, D_FEAT), jnp.float32),
            pltpu.VMEM((CHUNK, D_FEAT), jnp.float32),
            pltpu.VMEM((CHUNK, D_FEAT), jnp.float32),
            pltpu.VMEM((CHUNK, D_FEAT), jnp.float32),
            pltpu.VMEM((CHUNK, D_FEAT), jnp.float32),
            pltpu.VMEM((CHUNK,), jnp.float32),
            pltpu.VMEM((16,), jnp.float32),
            pltpu.VMEM((16,), jnp.float32),
            pltpu.SemaphoreType.DMA,
            pltpu.SemaphoreType.DMA,
        ],
    )
    def sc_loss(tab_hbm, left_hbm, pos_hbm, neg_hbm, gam_hbm, out_hbm,
                lidx, pidx, nidx, lrow0, prow0, nrow0, lrow1, prow1, nrow1,
                marg, gv, resv, sem0, sem1):
        cid = lax.axis_index("c")
        sid = lax.axis_index("s")
        wid = sid * NC + cid
        base = wid * P_W

        pltpu.sync_copy(left_hbm.at[pl.ds(base, P_W)], lidx)
        pltpu.sync_copy(pos_hbm.at[pl.ds(base, P_W)], pidx)
        pltpu.sync_copy(neg_hbm.at[pl.ds(base, P_W)], nidx)
        pltpu.sync_copy(gam_hbm, gv)
        g16 = gv[...]
        zero16 = jnp.zeros((16,), jnp.float32)

        bufs = ((lrow0, prow0, nrow0, sem0), (lrow1, prow1, nrow1, sem1))

        def start(b, ci):
            lr, pr, nr, sem = bufs[b]
            off = ci * CHUNK
            pltpu.async_copy(tab_hbm.at[lidx.at[pl.ds(off, CHUNK)]], lr, sem)
            pltpu.async_copy(tab_hbm.at[pidx.at[pl.ds(off, CHUNK)]], pr, sem)
            pltpu.async_copy(tab_hbm.at[nidx.at[pl.ds(off, CHUNK)]], nr, sem)

        def wait(b):
            lr, pr, nr, sem = bufs[b]
            for dst in (lr, pr, nr):
                pltpu.make_async_copy(tab_hbm.at[pl.ds(0, CHUNK)], dst,
                                      sem).wait()

        def compute(b, loss):
            lr, pr, nr, _ = bufs[b]

            for blk in range(BLKS):
                marg[pl.ds(blk * 16, 16)] = zero16

            @plsc.parallel_loop(0, CHUNK, unroll=4)
            def pair_body(p):
                acc = zero16
                for c in range(D_FEAT // 16):
                    sl = pl.ds(c * 16, 16)
                    acc = acc + lr[p, sl] * (nr[p, sl] - pr[p, sl])
                pidx16 = jnp.full((16,), p, jnp.int32)
                plsc.addupdate_scatter(marg, [pidx16], acc)

            for blk in range(BLKS):
                mv = marg[pl.ds(blk * 16, 16)]
                loss = loss + jnp.maximum(mv + g16, 0.0)
            return loss

        start(0, 0)

        def body(i, loss):
            c0 = 2 * i
            start(1, c0 + 1)
            wait(0)
            loss = compute(0, loss)
            start(0, c0 + 2)
            wait(1)
            return compute(1, loss)

        loss = lax.fori_loop(0, (NCHUNK - 1) // 2, body, zero16)
        wait(0)
        loss = compute(0, loss)
        resv[...] = loss
        pltpu.sync_copy(resv, out_hbm.at[wid])

    return sc_loss


_sc_loss = _make_sc_kernel()


def kernel(out, left, pos_right, neg_right, single_gamma):
    left = left.astype(jnp.int32)
    pos_right = pos_right.astype(jnp.int32)
    neg_right = neg_right.astype(jnp.int32)
    gam = jnp.full((16,), single_gamma, jnp.float32)
    partials = _sc_loss(out, left, pos_right, neg_right, gam)
    return jnp.sum(partials) / left.shape[0]


# speedup vs baseline: 1.4000x; 1.4000x over previous
import functools

import jax
import jax.numpy as jnp
from jax import lax
from jax.experimental import pallas as pl
from jax.experimental.pallas import tpu as pltpu
from jax.experimental.pallas import tpu_sc as plsc

N_NODES = 10000
D_FEAT = 128
N_PAIRS = 320000

NC = 2
NS = 16
NW = NC * NS
P_W = N_PAIRS // NW
CHUNK = 80
NCHUNK = P_W // CHUNK
BLKS = CHUNK // 16


def _make_sc_kernel():
    mesh = plsc.VectorSubcoreMesh(core_axis_name="c", subcore_axis_name="s")

    @functools.partial(
        pl.kernel,
        mesh=mesh,
        compiler_params=pltpu.CompilerParams(needs_layout_passes=False),
        out_type=jax.ShapeDtypeStruct((NW, 16), jnp.float32),
        scratch_types=[
            pltpu.VMEM((P_W,), jnp.int32),
            pltpu.VMEM((P_W,), jnp.int32),
            pltpu.VMEM((P_W,), jnp.int32),
            pltpu.VMEM((CHUNK, D_FEAT), jnp.float32),
            pltpu.VMEM((CHUNK, D_FEAT), jnp.float32),
            pltpu.VMEM((CHUNK, D_FEAT), jnp.float32),
            pltpu.VMEM((CHUNK, D_FEAT), jnp.float32),
            pltpu.VMEM((CHUNK, D_FEAT), jnp.float32),
            pltpu.VMEM((CHUNK, D_FEAT), jnp.float32),
            pltpu.VMEM((16,), jnp.float32),
            pltpu.VMEM((16,), jnp.float32),
            pltpu.SemaphoreType.DMA,
            pltpu.SemaphoreType.DMA,
        ],
    )
    def sc_loss(tab_hbm, left_hbm, pos_hbm, neg_hbm, gam_hbm, out_hbm,
                lidx, pidx, nidx, lrow0, prow0, nrow0, lrow1, prow1, nrow1,
                gv, resv, sem0, sem1):
        cid = lax.axis_index("c")
        sid = lax.axis_index("s")
        wid = sid * NC + cid
        base = wid * P_W

        pltpu.sync_copy(left_hbm.at[pl.ds(base, P_W)], lidx)
        pltpu.sync_copy(pos_hbm.at[pl.ds(base, P_W)], pidx)
        pltpu.sync_copy(neg_hbm.at[pl.ds(base, P_W)], nidx)
        pltpu.sync_copy(gam_hbm, gv)
        g16 = gv[...]
        zero16 = jnp.zeros((16,), jnp.float32)

        bufs = ((lrow0, prow0, nrow0, sem0), (lrow1, prow1, nrow1, sem1))

        def start(b, ci):
            lr, pr, nr, sem = bufs[b]
            off = ci * CHUNK
            pltpu.async_copy(tab_hbm.at[lidx.at[pl.ds(off, CHUNK)]], lr, sem)
            pltpu.async_copy(tab_hbm.at[pidx.at[pl.ds(off, CHUNK)]], pr, sem)
            pltpu.async_copy(tab_hbm.at[nidx.at[pl.ds(off, CHUNK)]], nr, sem)

        def wait(b):
            lr, pr, nr, sem = bufs[b]
            for dst in (lr, pr, nr):
                pltpu.make_async_copy(tab_hbm.at[pl.ds(0, CHUNK)], dst,
                                      sem).wait()

        g0 = g16[0]

        def compute(b, loss):
            lr, pr, nr, _ = bufs[b]

            def pair_body(p, loss):
                acc = zero16
                for c in range(D_FEAT // 16):
                    sl = pl.ds(c * 16, 16)
                    acc = acc + lr[p, sl] * (nr[p, sl] - pr[p, sl])
                m = jnp.sum(acc) + g0
                return loss + jnp.maximum(m, 0.0)

            return plsc.parallel_loop(0, CHUNK, carry=loss,
                                      unroll=4)(pair_body)

        start(0, 0)

        def body(i, loss):
            c0 = 2 * i
            start(1, c0 + 1)
            wait(0)
            loss = compute(0, loss)
            start(0, c0 + 2)
            wait(1)
            return compute(1, loss)

        loss = lax.fori_loop(0, (NCHUNK - 1) // 2, body, jnp.float32(0.0))
        wait(0)
        loss = compute(0, loss)
        resv[...] = jnp.full((16,), loss, jnp.float32)
        pltpu.sync_copy(resv, out_hbm.at[wid])

    return sc_loss


_sc_loss = _make_sc_kernel()


def kernel(out, left, pos_right, neg_right, single_gamma):
    left = left.astype(jnp.int32)
    pos_right = pos_right.astype(jnp.int32)
    neg_right = neg_right.astype(jnp.int32)
    gam = jnp.full((16,), single_gamma, jnp.float32)
    partials = _sc_loss(out, left, pos_right, neg_right, gam)
    return jnp.sum(partials[:, 0]) / left.shape[0]


# speedup vs baseline: 1.8034x; 1.2881x over previous
import functools

import jax
import jax.numpy as jnp
from jax import lax
from jax.experimental import pallas as pl
from jax.experimental.pallas import tpu as pltpu
from jax.experimental.pallas import tpu_sc as plsc

N_NODES = 10000
D_FEAT = 128
N_PAIRS = 320000

NC = 2
NS = 16
NW = NC * NS
P_W = N_PAIRS // NW
CHUNK = 80
NCHUNK = P_W // CHUNK
BLKS = CHUNK // 16


def _make_sc_kernel():
    mesh = plsc.VectorSubcoreMesh(core_axis_name="c", subcore_axis_name="s")

    @functools.partial(
        pl.kernel,
        mesh=mesh,
        compiler_params=pltpu.CompilerParams(needs_layout_passes=False,
                                             use_tc_tiling_on_sc=False),
        out_type=jax.ShapeDtypeStruct((NW, 16), jnp.float32),
        scratch_types=[
            pltpu.VMEM((P_W,), jnp.int32),
            pltpu.VMEM((P_W,), jnp.int32),
            pltpu.VMEM((P_W,), jnp.int32),
            pltpu.VMEM((CHUNK, D_FEAT // 2), jnp.int32),
            pltpu.VMEM((CHUNK, D_FEAT // 2), jnp.int32),
            pltpu.VMEM((CHUNK, D_FEAT // 2), jnp.int32),
            pltpu.VMEM((CHUNK, D_FEAT // 2), jnp.int32),
            pltpu.VMEM((CHUNK, D_FEAT // 2), jnp.int32),
            pltpu.VMEM((CHUNK, D_FEAT // 2), jnp.int32),
            pltpu.VMEM((16,), jnp.float32),
            pltpu.VMEM((16,), jnp.float32),
            pltpu.SemaphoreType.DMA,
            pltpu.SemaphoreType.DMA,
        ],
    )
    def sc_loss(tab_hbm, left_hbm, pos_hbm, neg_hbm, gam_hbm, out_hbm,
                lidx, pidx, nidx, lrow0, prow0, nrow0, lrow1, prow1, nrow1,
                gv, resv, sem0, sem1):
        cid = lax.axis_index("c")
        sid = lax.axis_index("s")
        wid = sid * NC + cid
        base = wid * P_W

        pltpu.sync_copy(left_hbm.at[pl.ds(base, P_W)], lidx)
        pltpu.sync_copy(pos_hbm.at[pl.ds(base, P_W)], pidx)
        pltpu.sync_copy(neg_hbm.at[pl.ds(base, P_W)], nidx)
        pltpu.sync_copy(gam_hbm, gv)
        g16 = gv[...]
        zero16 = jnp.zeros((16,), jnp.float32)

        bufs = ((lrow0, prow0, nrow0, sem0), (lrow1, prow1, nrow1, sem1))

        def start(b, ci):
            lr, pr, nr, sem = bufs[b]
            off = ci * CHUNK
            pltpu.async_copy(tab_hbm.at[lidx.at[pl.ds(off, CHUNK)]], lr, sem)
            pltpu.async_copy(tab_hbm.at[pidx.at[pl.ds(off, CHUNK)]], pr, sem)
            pltpu.async_copy(tab_hbm.at[nidx.at[pl.ds(off, CHUNK)]], nr, sem)

        def wait(b):
            lr, pr, nr, sem = bufs[b]
            for dst in (lr, pr, nr):
                pltpu.make_async_copy(tab_hbm.at[pl.ds(0, CHUNK)], dst,
                                      sem).wait()

        g0 = g16[0]

        def compute(b, loss):
            lr, pr, nr, _ = bufs[b]

            def pair_body(p, loss):
                acc_a = zero16
                acc_b = zero16
                for c in range(D_FEAT // 32):
                    sl = pl.ds(c * 16, 16)
                    l32 = plsc.bitcast(lr[p, sl], jnp.bfloat16)
                    d32 = (plsc.bitcast(nr[p, sl], jnp.bfloat16)
                           - plsc.bitcast(pr[p, sl], jnp.bfloat16))
                    la, lb = plsc.unpack(
                        l32, format=plsc.PackFormat.INTERLEAVED,
                        preferred_element_type=jnp.float32)
                    da, db = plsc.unpack(
                        d32, format=plsc.PackFormat.INTERLEAVED,
                        preferred_element_type=jnp.float32)
                    acc_a = acc_a + la * da
                    acc_b = acc_b + lb * db
                m = jnp.sum(acc_a + acc_b) + g0
                return loss + jnp.maximum(m, 0.0)

            return plsc.parallel_loop(0, CHUNK, carry=loss,
                                      unroll=4)(pair_body)

        start(0, 0)

        def body(i, loss):
            c0 = 2 * i
            start(1, c0 + 1)
            wait(0)
            loss = compute(0, loss)
            start(0, c0 + 2)
            wait(1)
            return compute(1, loss)

        loss = lax.fori_loop(0, (NCHUNK - 1) // 2, body, jnp.float32(0.0))
        wait(0)
        loss = compute(0, loss)
        resv[...] = jnp.full((16,), loss, jnp.float32)
        pltpu.sync_copy(resv, out_hbm.at[wid])

    return sc_loss


_sc_loss = _make_sc_kernel()


def kernel(out, left, pos_right, neg_right, single_gamma):
    out = lax.bitcast_convert_type(
        out.astype(jnp.bfloat16).reshape(N_NODES, D_FEAT // 2, 2),
        jnp.int32)
    left = left.astype(jnp.int32)
    pos_right = pos_right.astype(jnp.int32)
    neg_right = neg_right.astype(jnp.int32)
    gam = jnp.full((16,), single_gamma, jnp.float32)
    partials = _sc_loss(out, left, pos_right, neg_right, gam)
    return jnp.sum(partials[:, 0]) / left.shape[0]


# speedup vs baseline: 1.8128x; 1.0052x over previous
import functools

import jax
import jax.numpy as jnp
from jax import lax
from jax.experimental import pallas as pl
from jax.experimental.pallas import tpu as pltpu
from jax.experimental.pallas import tpu_sc as plsc

N_NODES = 10000
D_FEAT = 128
N_PAIRS = 320000

NC = 2
NS = 16
NW = NC * NS
P_W = N_PAIRS // NW
CHUNK = 80
NCHUNK = P_W // CHUNK
BLKS = CHUNK // 16


def _make_sc_kernel():
    mesh = plsc.VectorSubcoreMesh(core_axis_name="c", subcore_axis_name="s")

    @functools.partial(
        pl.kernel,
        mesh=mesh,
        compiler_params=pltpu.CompilerParams(needs_layout_passes=False,
                                             use_tc_tiling_on_sc=False),
        out_type=jax.ShapeDtypeStruct((NW, 16), jnp.float32),
        scratch_types=[
            pltpu.VMEM((P_W,), jnp.int32),
            pltpu.VMEM((P_W,), jnp.int32),
            pltpu.VMEM((P_W,), jnp.int32),
            pltpu.VMEM((CHUNK, D_FEAT // 2), jnp.int32),
            pltpu.VMEM((CHUNK, D_FEAT // 2), jnp.int32),
            pltpu.VMEM((CHUNK, D_FEAT // 2), jnp.int32),
            pltpu.VMEM((CHUNK, D_FEAT // 2), jnp.int32),
            pltpu.VMEM((CHUNK, D_FEAT // 2), jnp.int32),
            pltpu.VMEM((CHUNK, D_FEAT // 2), jnp.int32),
            pltpu.VMEM((16,), jnp.float32),
            pltpu.VMEM((16,), jnp.float32),
            pltpu.SemaphoreType.DMA,
            pltpu.SemaphoreType.DMA,
        ],
    )
    def sc_loss(tab_hbm, left_hbm, pos_hbm, neg_hbm, gam_hbm, out_hbm,
                lidx, pidx, nidx, lrow0, prow0, nrow0, lrow1, prow1, nrow1,
                gv, resv, sem0, sem1):
        cid = lax.axis_index("c")
        sid = lax.axis_index("s")
        wid = sid * NC + cid
        base = wid * P_W

        pltpu.sync_copy(left_hbm.at[pl.ds(base, P_W)], lidx)
        pltpu.sync_copy(pos_hbm.at[pl.ds(base, P_W)], pidx)
        pltpu.sync_copy(neg_hbm.at[pl.ds(base, P_W)], nidx)
        pltpu.sync_copy(gam_hbm, gv)
        g16 = gv[...]
        zero16 = jnp.zeros((16,), jnp.float32)

        bufs = ((lrow0, prow0, nrow0, sem0), (lrow1, prow1, nrow1, sem1))

        def start(b, ci):
            lr, pr, nr, sem = bufs[b]
            off = ci * CHUNK
            pltpu.async_copy(tab_hbm.at[lidx.at[pl.ds(off, CHUNK)]], lr, sem)
            pltpu.async_copy(tab_hbm.at[pidx.at[pl.ds(off, CHUNK)]], pr, sem)
            pltpu.async_copy(tab_hbm.at[nidx.at[pl.ds(off, CHUNK)]], nr, sem)

        def wait(b):
            lr, pr, nr, sem = bufs[b]
            for dst in (lr, pr, nr):
                pltpu.make_async_copy(tab_hbm.at[pl.ds(0, CHUNK)], dst,
                                      sem).wait()

        g0 = g16[0]

        def compute(b, loss):
            lr, pr, nr, _ = bufs[b]

            def pair_body(p, loss):
                acc_a = zero16
                acc_b = zero16
                for c in range(D_FEAT // 32):
                    sl = pl.ds(c * 16, 16)
                    l32 = plsc.bitcast(lr[p, sl], jnp.bfloat16)
                    d32 = (plsc.bitcast(nr[p, sl], jnp.bfloat16)
                           - plsc.bitcast(pr[p, sl], jnp.bfloat16))
                    prod = l32 * d32
                    pa, pb = plsc.unpack(
                        prod, format=plsc.PackFormat.INTERLEAVED,
                        preferred_element_type=jnp.float32)
                    acc_a = acc_a + pa
                    acc_b = acc_b + pb
                m = jnp.sum(acc_a + acc_b) + g0
                return loss + jnp.maximum(m, 0.0)

            return plsc.parallel_loop(0, CHUNK, carry=loss,
                                      unroll=4)(pair_body)

        start(0, 0)

        def body(i, loss):
            c0 = 2 * i
            start(1, c0 + 1)
            wait(0)
            loss = compute(0, loss)
            start(0, c0 + 2)
            wait(1)
            return compute(1, loss)

        loss = lax.fori_loop(0, (NCHUNK - 1) // 2, body, jnp.float32(0.0))
        wait(0)
        loss = compute(0, loss)
        resv[...] = jnp.full((16,), loss, jnp.float32)
        pltpu.sync_copy(resv, out_hbm.at[wid])

    return sc_loss


_sc_loss = _make_sc_kernel()


def kernel(out, left, pos_right, neg_right, single_gamma):
    out = lax.bitcast_convert_type(
        out.astype(jnp.bfloat16).reshape(N_NODES, D_FEAT // 2, 2),
        jnp.int32)
    left = left.astype(jnp.int32)
    pos_right = pos_right.astype(jnp.int32)
    neg_right = neg_right.astype(jnp.int32)
    gam = jnp.full((16,), single_gamma, jnp.float32)
    partials = _sc_loss(out, left, pos_right, neg_right, gam)
    return jnp.sum(partials[:, 0]) / left.shape[0]


# speedup vs baseline: 2.2242x; 1.2270x over previous
import functools

import jax
import jax.numpy as jnp
from jax import lax
from jax.experimental import pallas as pl
from jax.experimental.pallas import tpu as pltpu
from jax.experimental.pallas import tpu_sc as plsc

N_NODES = 10000
D_FEAT = 128
N_PAIRS = 320000

NC = 2
NS = 16
NW = NC * NS
P_W = N_PAIRS // NW
CHUNK = 80
NCHUNK = P_W // CHUNK
BLKS = CHUNK // 16


def _make_sc_kernel():
    mesh = plsc.VectorSubcoreMesh(core_axis_name="c", subcore_axis_name="s")

    @functools.partial(
        pl.kernel,
        mesh=mesh,
        compiler_params=pltpu.CompilerParams(needs_layout_passes=False,
                                             use_tc_tiling_on_sc=False),
        out_type=jax.ShapeDtypeStruct((NW, 16), jnp.float32),
        scratch_types=[
            pltpu.VMEM((P_W,), jnp.int32),
            pltpu.VMEM((P_W,), jnp.int32),
            pltpu.VMEM((P_W,), jnp.int32),
            pltpu.VMEM((CHUNK, D_FEAT // 2), jnp.int32),
            pltpu.VMEM((CHUNK, D_FEAT // 2), jnp.int32),
            pltpu.VMEM((CHUNK, D_FEAT // 2), jnp.int32),
            pltpu.VMEM((CHUNK, D_FEAT // 2), jnp.int32),
            pltpu.VMEM((CHUNK, D_FEAT // 2), jnp.int32),
            pltpu.VMEM((CHUNK, D_FEAT // 2), jnp.int32),
            pltpu.VMEM((16,), jnp.float32),
            pltpu.VMEM((16,), jnp.float32),
            pltpu.VMEM_SHARED((N_NODES, D_FEAT // 2), jnp.int32),
            pltpu.SemaphoreType.DMA,
            pltpu.SemaphoreType.DMA,
        ],
    )
    def sc_loss(tab_hbm, left_hbm, pos_hbm, neg_hbm, gam_hbm, out_hbm,
                lidx, pidx, nidx, lrow0, prow0, nrow0, lrow1, prow1, nrow1,
                gv, resv, stab, sem0, sem1):
        cid = lax.axis_index("c")
        sid = lax.axis_index("s")
        wid = sid * NC + cid
        base = wid * P_W

        pltpu.sync_copy(left_hbm.at[pl.ds(base, P_W)], lidx)
        pltpu.sync_copy(pos_hbm.at[pl.ds(base, P_W)], pidx)
        pltpu.sync_copy(neg_hbm.at[pl.ds(base, P_W)], nidx)
        pltpu.sync_copy(gam_hbm, gv)

        rows_per_sub = N_NODES // NS
        pltpu.sync_copy(tab_hbm.at[pl.ds(sid * rows_per_sub, rows_per_sub)],
                        stab.at[pl.ds(sid * rows_per_sub, rows_per_sub)])
        plsc.subcore_barrier()
        g16 = gv[...]
        zero16 = jnp.zeros((16,), jnp.float32)

        bufs = ((lrow0, prow0, nrow0, sem0), (lrow1, prow1, nrow1, sem1))

        def start(b, ci):
            lr, pr, nr, sem = bufs[b]
            off = ci * CHUNK
            pltpu.async_copy(stab.at[lidx.at[pl.ds(off, CHUNK)]], lr, sem)
            pltpu.async_copy(stab.at[pidx.at[pl.ds(off, CHUNK)]], pr, sem)
            pltpu.async_copy(stab.at[nidx.at[pl.ds(off, CHUNK)]], nr, sem)

        def wait(b):
            lr, pr, nr, sem = bufs[b]
            for dst in (lr, pr, nr):
                pltpu.make_async_copy(tab_hbm.at[pl.ds(0, CHUNK)], dst,
                                      sem).wait()

        g0 = g16[0]

        def compute(b, loss):
            lr, pr, nr, _ = bufs[b]

            def pair_body(p, loss):
                acc_a = zero16
                acc_b = zero16
                for c in range(D_FEAT // 32):
                    sl = pl.ds(c * 16, 16)
                    l32 = plsc.bitcast(lr[p, sl], jnp.bfloat16)
                    d32 = (plsc.bitcast(nr[p, sl], jnp.bfloat16)
                           - plsc.bitcast(pr[p, sl], jnp.bfloat16))
                    prod = l32 * d32
                    pa, pb = plsc.unpack(
                        prod, format=plsc.PackFormat.INTERLEAVED,
                        preferred_element_type=jnp.float32)
                    acc_a = acc_a + pa
                    acc_b = acc_b + pb
                m = jnp.sum(acc_a + acc_b) + g0
                return loss + jnp.maximum(m, 0.0)

            return plsc.parallel_loop(0, CHUNK, carry=loss,
                                      unroll=4)(pair_body)

        start(0, 0)

        def body(i, loss):
            c0 = 2 * i
            start(1, c0 + 1)
            wait(0)
            loss = compute(0, loss)
            start(0, c0 + 2)
            wait(1)
            return compute(1, loss)

        loss = lax.fori_loop(0, (NCHUNK - 1) // 2, body, jnp.float32(0.0))
        wait(0)
        loss = compute(0, loss)
        resv[...] = jnp.full((16,), loss, jnp.float32)
        pltpu.sync_copy(resv, out_hbm.at[wid])

    return sc_loss


_sc_loss = _make_sc_kernel()


def kernel(out, left, pos_right, neg_right, single_gamma):
    out = lax.bitcast_convert_type(
        out.astype(jnp.bfloat16).reshape(N_NODES, D_FEAT // 2, 2),
        jnp.int32)
    left = left.astype(jnp.int32)
    pos_right = pos_right.astype(jnp.int32)
    neg_right = neg_right.astype(jnp.int32)
    gam = jnp.full((16,), single_gamma, jnp.float32)
    partials = _sc_loss(out, left, pos_right, neg_right, gam)
    return jnp.sum(partials[:, 0]) / left.shape[0]
